# Initial kernel scaffold; baseline (speedup 1.0000x reference)
import jax
import jax.numpy as jnp
from jax.experimental import pallas as pl


def kernel(x, w1, b1, p1, w2, b2, p2, wf1, bf1, wf2, bf2, wf3, bf3):
    raise NotImplementedError("write your pallas kernel here")


# speedup vs baseline: 1.2158x; 1.2158x over previous
import jax
import jax.numpy as jnp
from jax.experimental import pallas as pl
from jax.experimental.pallas import tpu as pltpu

_K = 5
_CIN_PAD = 8
_W1 = 32
_IN_ROWS = 1024
_CONV1_N = 32
_POOL1_ROWS = 224
_S2 = 16
_CONV2_ROWS = 160
_COUT2 = 128
_FC_SP = 25
_FC_H = 128


def _fused_body(x_ref, w1_ref, b1_ref, p1_ref, w2_ref, b2_ref, pm_ref,
                wf1_ref, bf1_ref, wf2_ref, bf2_ref, wf3_ref, bf3_ref,
                o_ref, *, bb):
    f32, bf16 = jnp.float32, jnp.bfloat16
    n1 = bb * _IN_ROWS
    m1 = n1 - (_K - 1) * _W1
    m2 = bb * _POOL1_ROWS
    m2v = m2 - (_K - 1) * _S2

    xa = x_ref[...].reshape(n1, _K * _CIN_PAD)
    acc1 = None
    for di in range(_K):
        t = jnp.dot(xa[di * _W1:di * _W1 + m1, :],
                    w1_ref[pl.ds(di * _K * _CIN_PAD, _K * _CIN_PAD), :],
                    preferred_element_type=f32)
        acc1 = t if acc1 is None else acc1 + t
    a1 = jnp.tanh(acc1 + b1_ref[...]).astype(bf16)

    p1t = p1_ref[...]
    pooled1 = jnp.concatenate(
        [jnp.dot(p1t, a1[bi * _IN_ROWS:bi * _IN_ROWS + (_W1 - _K + 1) * _W1, :],
                 preferred_element_type=f32).astype(bf16)
         for bi in range(bb)], axis=0)

    pooled1p = jnp.concatenate(
        [pooled1, jnp.zeros((_S2, _CONV1_N), bf16)], axis=0)
    x2 = jnp.concatenate(
        [pooled1p[dj:dj + m2, :] for dj in range(_K)], axis=1)
    acc2 = None
    for di in range(_K):
        t = jnp.dot(x2[di * _S2:di * _S2 + m2v, :],
                    w2_ref[pl.ds(di * _K * _CONV1_N, _K * _CONV1_N), :],
                    preferred_element_type=f32)
        acc2 = t if acc2 is None else acc2 + t
    a2 = jnp.tanh(acc2 + b2_ref[...]).astype(bf16)

    pooled2 = jnp.dot(pm_ref[...], a2, preferred_element_type=f32)

    h = None
    for p in range(_FC_SP):
        t = jnp.dot(pooled2[p * bb:(p + 1) * bb, :].astype(bf16), wf1_ref[p],
                    preferred_element_type=f32)
        h = t if h is None else h + t
    h = jnp.tanh(h + bf1_ref[...]).astype(bf16)

    h = jnp.tanh(jnp.dot(h, wf2_ref[...], preferred_element_type=f32)
                 + bf2_ref[...]).astype(bf16)
    o = jnp.dot(h, wf3_ref[...], preferred_element_type=f32) + bf3_ref[...]
    o_ref[...] = o.reshape(bb, 1, 2)


def kernel(x, w1, b1, p1, w2, b2, p2, wf1, bf1, wf2, bf2, wf3, bf3):
    B = x.shape[0]
    bb = 1
    for cand in (8, 4, 2):
        if B % cand == 0 and B // cand >= 2:
            bb = cand
            break

    xt = jnp.transpose(x, (0, 2, 3, 1))
    xt = jnp.pad(xt, ((0, 0), (0, 0), (0, 0), (0, _CIN_PAD - 3)))
    xt = xt.reshape(B, _IN_ROWS, _CIN_PAD).astype(jnp.bfloat16)
    xt = jnp.pad(xt, ((0, 0), (0, _K - 1), (0, 0)))
    xt = jnp.concatenate([xt[:, dj:dj + _IN_ROWS, :] for dj in range(_K)],
                         axis=-1)

    p1t = p1[:_POOL1_ROWS]

    m2v = bb * _POOL1_ROWS - (_K - 1) * _S2
    p2p = jnp.pad(p2, ((0, 0), (0, _POOL1_ROWS - _CONV2_ROWS)))
    eye = jnp.eye(bb, dtype=p2.dtype)
    pm = (eye[None, :, :, None] * p2p[:, None, None, :])
    pm = pm.reshape(32 * bb, bb * _POOL1_ROWS)[:, :m2v]

    import functools
    body = functools.partial(_fused_body, bb=bb)
    c2 = lambda i: (0, 0)
    c3 = lambda i: (0, 0, 0)

    out = pl.pallas_call(
        body,
        grid=(B // bb,),
        out_shape=jax.ShapeDtypeStruct((B, 1, 2), jnp.float32),
        in_specs=[
            pl.BlockSpec((bb, _IN_ROWS, _K * _CIN_PAD), lambda i: (i, 0, 0)),
            pl.BlockSpec((_K * _K * _CIN_PAD, _CONV1_N), c2),
            pl.BlockSpec((1, _CONV1_N), c2),
            pl.BlockSpec((_POOL1_ROWS, 896), c2),
            pl.BlockSpec((_K * _K * _CONV1_N, _COUT2), c2),
            pl.BlockSpec((1, _COUT2), c2),
            pl.BlockSpec((32 * bb, m2v), c2),
            pl.BlockSpec((_FC_SP, _FC_H, _FC_H), c3),
            pl.BlockSpec((1, _FC_H), c2),
            pl.BlockSpec((_FC_H, _FC_H), c2),
            pl.BlockSpec((1, _FC_H), c2),
            pl.BlockSpec((_FC_H, 2), c2),
            pl.BlockSpec((1, 2), c2),
        ],
        out_specs=pl.BlockSpec((bb, 1, 2), lambda i: (i, 0, 0)),
        compiler_params=pltpu.CompilerParams(
            dimension_semantics=("parallel",)),
    )(xt, w1, b1, p1t, w2, b2, pm,
      wf1, bf1, wf2, bf2, wf3, bf3)
    return out.reshape(B, 2)


# speedup vs baseline: 9.4643x; 7.7846x over previous
import functools

import jax
import jax.numpy as jnp
from jax.experimental import pallas as pl
from jax.experimental.pallas import tpu as pltpu

_K = 5
_CIN = 3
_SP = 1024
_LANE_PAD = 1152 + 32
_CONV1_N = 32
_P1_N = 224
_S2 = 16
_CONV2_L = 160
_X2_SPAN = 228
_COUT2 = 128
_FC_SP = 25
_FC_H = 128


def _fused_body(x_ref, w1_ref, b1t_ref, p1_ref, w2_ref, b2t_ref, pm_ref,
                wf1_ref, bf1_ref, wf2_ref, bf2_ref, wf3_ref, bf3_ref,
                o_ref, *, bb):
    f32, bf16 = jnp.float32, jnp.bfloat16

    w1t = w1_ref[...]
    b1t = b1t_ref[...]
    a1_list = []
    for bi in range(bb):
        xi = x_ref[bi].astype(bf16)
        xip = jnp.concatenate(
            [xi, jnp.zeros((_CIN, _LANE_PAD - _SP), bf16)], axis=1)
        xd = jnp.concatenate(
            [xip[:, dj:dj + 1152] for dj in range(_K)], axis=0)
        xdp = jnp.concatenate([xd, jnp.zeros((1, 1152), bf16)], axis=0)
        xt = jnp.concatenate(
            [xdp[:, di * 32:di * 32 + _SP] for di in range(_K)],
            axis=0)
        acc = jnp.dot(w1t, xt, preferred_element_type=f32)
        a1_list.append(jnp.tanh(acc + b1t).astype(bf16))

    p1 = p1_ref[...]
    pooled1 = [
        jnp.concatenate(
            [jnp.dot(a1, p1, preferred_element_type=f32).astype(bf16),
             jnp.zeros((_CONV1_N, 256 - _P1_N), bf16)], axis=1)
        for a1 in a1_list]

    w2t = w2_ref[...]
    b2t = b2t_ref[...]
    a2_list = []
    for bi in range(bb):
        x2 = jnp.concatenate(
            [pooled1[bi][:, di * _S2 + dj:di * _S2 + dj + _CONV2_L]
             for di in range(_K) for dj in range(_K)], axis=0)
        acc = jnp.dot(w2t, x2, preferred_element_type=f32)
        a2_list.append(jnp.tanh(acc + b2t).astype(bf16))

    a2_all = jnp.concatenate(a2_list, axis=1)
    pooled2 = jnp.dot(a2_all, pm_ref[...],
                      preferred_element_type=f32)
    pooled2 = jnp.transpose(pooled2)

    h = None
    for p in range(_FC_SP):
        t = jnp.dot(pooled2[p * bb:(p + 1) * bb, :].astype(bf16), wf1_ref[p],
                    preferred_element_type=f32)
        h = t if h is None else h + t
    h = jnp.tanh(h + bf1_ref[...]).astype(bf16)

    h = jnp.tanh(jnp.dot(h, wf2_ref[...], preferred_element_type=f32)
                 + bf2_ref[...]).astype(bf16)
    o = jnp.dot(h, wf3_ref[...], preferred_element_type=f32) + bf3_ref[...]
    o_ref[...] = o.reshape(bb, 1, 2)


def kernel(x, w1, b1, p1, w2, b2, p2, wf1, bf1, wf2, bf2, wf3, bf3):
    B = x.shape[0]
    bb = 1
    for cand in (8, 4, 2):
        if B % cand == 0 and B // cand >= 2:
            bb = cand
            break
    bf16 = jnp.bfloat16

    xr = x.reshape(B, _CIN, _SP)

    w1r = w1.reshape(_K, _K, 8, _CONV1_N)[:, :, :_CIN, :]
    w1t = jnp.transpose(w1r, (3, 0, 1, 2)).reshape(_CONV1_N, _K, _K * _CIN)
    w1t = jnp.pad(w1t, ((0, 0), (0, 0), (0, 1))).reshape(_CONV1_N, _K * 16)
    b1t = jnp.transpose(b1)

    p1t = jnp.pad(jnp.transpose(p1[:_P1_N]), ((0, _SP - 896), (0, 0)))
    p1t = p1t.astype(bf16)

    w2t = jnp.transpose(w2)
    b2t = jnp.transpose(b2)

    eye = jnp.eye(bb, dtype=p2.dtype)
    pm = (eye[:, None, None, :] * jnp.transpose(p2)[None, :, :, None])
    pm = pm.reshape(bb * _CONV2_L, 32 * bb)

    body = functools.partial(_fused_body, bb=bb)
    c2 = lambda i: (0, 0)
    c3 = lambda i: (0, 0, 0)

    out = pl.pallas_call(
        body,
        grid=(B // bb,),
        out_shape=jax.ShapeDtypeStruct((B, 1, 2), jnp.float32),
        in_specs=[
            pl.BlockSpec((bb, _CIN, _SP), lambda i: (i, 0, 0)),
            pl.BlockSpec((_CONV1_N, _K * 16), c2),
            pl.BlockSpec((_CONV1_N, 1), c2),
            pl.BlockSpec((_SP, _P1_N), c2),
            pl.BlockSpec((_COUT2, _K * _K * _CONV1_N), c2),
            pl.BlockSpec((_COUT2, 1), c2),
            pl.BlockSpec((bb * _CONV2_L, 32 * bb), c2),
            pl.BlockSpec((_FC_SP, _FC_H, _FC_H), c3),
            pl.BlockSpec((1, _FC_H), c2),
            pl.BlockSpec((_FC_H, _FC_H), c2),
            pl.BlockSpec((1, _FC_H), c2),
            pl.BlockSpec((_FC_H, 2), c2),
            pl.BlockSpec((1, 2), c2),
        ],
        out_specs=pl.BlockSpec((bb, 1, 2), lambda i: (i, 0, 0)),
        compiler_params=pltpu.CompilerParams(
            dimension_semantics=("parallel",)),
    )(xr, w1t, b1t, p1t, w2t, b2t, pm,
      wf1, bf1, wf2, bf2, wf3, bf3)
    return out.reshape(B, 2)


# speedup vs baseline: 9.7162x; 1.0266x over previous
import functools

import jax
import jax.numpy as jnp
from jax.experimental import pallas as pl
from jax.experimental.pallas import tpu as pltpu

_K = 5
_CIN = 3
_SP = 1024
_LANE_PAD = 1152 + 32
_CONV1_N = 32
_P1_N = 224
_S2 = 16
_CONV2_L = 160
_X2_SPAN = 228
_COUT2 = 128
_FC_SP = 25
_FC_H = 128


def _fused_body(x_ref, w1_ref, b1t_ref, p1_ref, w2_ref, b2t_ref, pm_ref,
                wf1_ref, bf1_ref, wf2_ref, bf2_ref, wf3_ref, bf3_ref,
                o_ref, *, bb):
    f32, bf16 = jnp.float32, jnp.bfloat16

    w1t = w1_ref[...]
    b1t = b1t_ref[...]
    xall = x_ref[...].astype(bf16)
    a1_list = []
    for bi in range(bb):
        xi = xall[bi]
        xip = jnp.concatenate(
            [xi, jnp.zeros((_CIN, _LANE_PAD - _SP), bf16)], axis=1)
        xd = jnp.concatenate(
            [xip[:, dj:dj + 1152] for dj in range(_K)], axis=0)
        xdp = jnp.concatenate([xd, jnp.zeros((1, 1152), bf16)], axis=0)
        xt = jnp.concatenate(
            [xdp[:, di * 32:di * 32 + _SP] for di in range(_K)],
            axis=0)
        acc = jnp.dot(w1t, xt, preferred_element_type=f32)
        a1_list.append(jnp.tanh(acc + b1t).astype(bf16))

    a1s = jnp.concatenate(a1_list, axis=0)
    pooled1 = jnp.dot(a1s, p1_ref[...],
                      preferred_element_type=f32).astype(bf16)
    pooled1 = jnp.concatenate(
        [pooled1, jnp.zeros((bb * _CONV1_N, 256 - _P1_N), bf16)],
        axis=1)

    x2 = jnp.concatenate(
        [jnp.concatenate(
            [pooled1[bi * _CONV1_N:(bi + 1) * _CONV1_N,
                     di * _S2 + dj:di * _S2 + dj + _CONV2_L]
             for bi in range(bb)], axis=1)
         for di in range(_K) for dj in range(_K)], axis=0)
    acc2 = jnp.dot(w2_ref[...], x2, preferred_element_type=f32)
    a2_all = jnp.tanh(acc2 + b2t_ref[...]).astype(bf16)

    pooled2 = jnp.dot(a2_all, pm_ref[...],
                      preferred_element_type=f32)
    pooled2 = jnp.transpose(pooled2)

    h = None
    for p in range(_FC_SP):
        t = jnp.dot(pooled2[p * bb:(p + 1) * bb, :].astype(bf16), wf1_ref[p],
                    preferred_element_type=f32)
        h = t if h is None else h + t
    h = jnp.tanh(h + bf1_ref[...]).astype(bf16)

    h = jnp.tanh(jnp.dot(h, wf2_ref[...], preferred_element_type=f32)
                 + bf2_ref[...]).astype(bf16)
    o = jnp.dot(h, wf3_ref[...], preferred_element_type=f32) + bf3_ref[...]
    o_ref[...] = o.reshape(bb, 1, 2)


def kernel(x, w1, b1, p1, w2, b2, p2, wf1, bf1, wf2, bf2, wf3, bf3):
    B = x.shape[0]
    bb = 1
    for cand in (8, 4, 2):
        if B % cand == 0 and B // cand >= 2:
            bb = cand
            break
    bf16 = jnp.bfloat16

    xr = x.reshape(B, _CIN, _SP)

    w1r = w1.reshape(_K, _K, 8, _CONV1_N)[:, :, :_CIN, :]
    w1t = jnp.transpose(w1r, (3, 0, 1, 2)).reshape(_CONV1_N, _K, _K * _CIN)
    w1t = jnp.pad(w1t, ((0, 0), (0, 0), (0, 1))).reshape(_CONV1_N, _K * 16)
    b1t = jnp.transpose(b1)

    p1t = jnp.pad(jnp.transpose(p1[:_P1_N]), ((0, _SP - 896), (0, 0)))
    p1t = p1t.astype(bf16)

    w2t = jnp.transpose(w2)
    b2t = jnp.transpose(b2)

    eye = jnp.eye(bb, dtype=p2.dtype)
    pm = (eye[:, None, None, :] * jnp.transpose(p2)[None, :, :, None])
    pm = pm.reshape(bb * _CONV2_L, 32 * bb)

    body = functools.partial(_fused_body, bb=bb)
    c2 = lambda i: (0, 0)
    c3 = lambda i: (0, 0, 0)

    out = pl.pallas_call(
        body,
        grid=(B // bb,),
        out_shape=jax.ShapeDtypeStruct((B, 1, 2), jnp.float32),
        in_specs=[
            pl.BlockSpec((bb, _CIN, _SP), lambda i: (i, 0, 0)),
            pl.BlockSpec((_CONV1_N, _K * 16), c2),
            pl.BlockSpec((_CONV1_N, 1), c2),
            pl.BlockSpec((_SP, _P1_N), c2),
            pl.BlockSpec((_COUT2, _K * _K * _CONV1_N), c2),
            pl.BlockSpec((_COUT2, 1), c2),
            pl.BlockSpec((bb * _CONV2_L, 32 * bb), c2),
            pl.BlockSpec((_FC_SP, _FC_H, _FC_H), c3),
            pl.BlockSpec((1, _FC_H), c2),
            pl.BlockSpec((_FC_H, _FC_H), c2),
            pl.BlockSpec((1, _FC_H), c2),
            pl.BlockSpec((_FC_H, 2), c2),
            pl.BlockSpec((1, 2), c2),
        ],
        out_specs=pl.BlockSpec((bb, 1, 2), lambda i: (i, 0, 0)),
        compiler_params=pltpu.CompilerParams(
            dimension_semantics=("parallel",)),
    )(xr, w1t, b1t, p1t, w2t, b2t, pm,
      wf1, bf1, wf2, bf2, wf3, bf3)
    return out.reshape(B, 2)


# speedup vs baseline: 11.4267x; 1.1760x over previous
import functools

import jax
import jax.numpy as jnp
from jax.experimental import pallas as pl
from jax.experimental.pallas import tpu as pltpu

_K = 5
_CIN = 3
_SP = 1024
_LANE_PAD = 1152 + 32
_CONV1_N = 32
_P1_N = 224
_S2 = 16
_CONV2_L = 160
_X2_SPAN = 228
_COUT2 = 128
_FC_SP = 25
_FC_H = 128


def _fused_body(x_ref, w1_ref, b1t_ref, p1_ref, w2_ref, b2t_ref, pm_ref,
                wf1_ref, bf1_ref, wf2_ref, bf2_ref, wf3_ref, bf3_ref,
                o_ref, *, bb):
    f32, bf16 = jnp.float32, jnp.bfloat16

    w1t = w1_ref[...]
    b1t = b1t_ref[...]
    xall = x_ref[...].astype(bf16)
    a1_list = []
    for bi in range(bb):
        xi = xall[bi]
        xip = jnp.concatenate(
            [xi, jnp.zeros((_CIN, _LANE_PAD - _SP), bf16)], axis=1)
        xd = jnp.concatenate(
            [xip[:, dj:dj + 1152] for dj in range(_K)], axis=0)
        xdp = jnp.concatenate([xd, jnp.zeros((1, 1152), bf16)], axis=0)
        xt = jnp.concatenate(
            [xdp[:, di * 32:di * 32 + _SP] for di in range(_K)],
            axis=0)
        acc = jnp.dot(w1t, xt, preferred_element_type=f32)
        a1_list.append(jnp.tanh(acc + b1t).astype(bf16))

    a1s = jnp.concatenate(a1_list, axis=0)
    pooled1 = jnp.dot(a1s, p1_ref[...],
                      preferred_element_type=f32).astype(bf16)
    pooled1 = jnp.concatenate(
        [pooled1, jnp.zeros((bb * _CONV1_N, 256 - _P1_N), bf16)],
        axis=1)

    x2 = jnp.concatenate(
        [jnp.concatenate(
            [pooled1[bi * _CONV1_N:(bi + 1) * _CONV1_N,
                     di * _S2 + dj:di * _S2 + dj + _CONV2_L]
             for bi in range(bb)], axis=1)
         for di in range(_K) for dj in range(_K)], axis=0)
    acc2 = jnp.dot(w2_ref[...], x2, preferred_element_type=f32)
    a2_all = jnp.tanh(acc2 + b2t_ref[...]).astype(bf16)

    pooled2 = jnp.dot(a2_all, pm_ref[...],
                      preferred_element_type=f32)
    pooled2 = jnp.transpose(pooled2)

    h = None
    for p in range(_FC_SP):
        t = jnp.dot(pooled2[p * bb:(p + 1) * bb, :].astype(bf16), wf1_ref[p],
                    preferred_element_type=f32)
        h = t if h is None else h + t
    h = jnp.tanh(h + bf1_ref[...]).astype(bf16)

    h = jnp.tanh(jnp.dot(h, wf2_ref[...], preferred_element_type=f32)
                 + bf2_ref[...]).astype(bf16)
    o = jnp.dot(h, wf3_ref[...], preferred_element_type=f32) + bf3_ref[...]
    o_ref[...] = o.reshape(bb, 1, 2)


def kernel(x, w1, b1, p1, w2, b2, p2, wf1, bf1, wf2, bf2, wf3, bf3):
    B = x.shape[0]
    bb = 1
    for cand in (16, 8, 4, 2):
        if B % cand == 0 and B // cand >= 2:
            bb = cand
            break
    bf16 = jnp.bfloat16

    xr = x.reshape(B, _CIN, _SP)

    w1r = w1.reshape(_K, _K, 8, _CONV1_N)[:, :, :_CIN, :]
    w1t = jnp.transpose(w1r, (3, 0, 1, 2)).reshape(_CONV1_N, _K, _K * _CIN)
    w1t = jnp.pad(w1t, ((0, 0), (0, 0), (0, 1))).reshape(_CONV1_N, _K * 16)
    b1t = jnp.transpose(b1)

    p1t = jnp.pad(jnp.transpose(p1[:_P1_N]), ((0, _SP - 896), (0, 0)))
    p1t = p1t.astype(bf16)

    w2t = jnp.transpose(w2)
    b2t = jnp.transpose(b2)

    eye = jnp.eye(bb, dtype=p2.dtype)
    pm = (eye[:, None, None, :] * jnp.transpose(p2)[None, :, :, None])
    pm = pm.reshape(bb * _CONV2_L, 32 * bb)

    body = functools.partial(_fused_body, bb=bb)
    c2 = lambda i: (0, 0)
    c3 = lambda i: (0, 0, 0)

    out = pl.pallas_call(
        body,
        grid=(B // bb,),
        out_shape=jax.ShapeDtypeStruct((B, 1, 2), jnp.float32),
        in_specs=[
            pl.BlockSpec((bb, _CIN, _SP), lambda i: (i, 0, 0)),
            pl.BlockSpec((_CONV1_N, _K * 16), c2),
            pl.BlockSpec((_CONV1_N, 1), c2),
            pl.BlockSpec((_SP, _P1_N), c2),
            pl.BlockSpec((_COUT2, _K * _K * _CONV1_N), c2),
            pl.BlockSpec((_COUT2, 1), c2),
            pl.BlockSpec((bb * _CONV2_L, 32 * bb), c2),
            pl.BlockSpec((_FC_SP, _FC_H, _FC_H), c3),
            pl.BlockSpec((1, _FC_H), c2),
            pl.BlockSpec((_FC_H, _FC_H), c2),
            pl.BlockSpec((1, _FC_H), c2),
            pl.BlockSpec((_FC_H, 2), c2),
            pl.BlockSpec((1, 2), c2),
        ],
        out_specs=pl.BlockSpec((bb, 1, 2), lambda i: (i, 0, 0)),
        compiler_params=pltpu.CompilerParams(
            dimension_semantics=("parallel",)),
    )(xr, w1t, b1t, p1t, w2t, b2t, pm,
      wf1, bf1, wf2, bf2, wf3, bf3)
    return out.reshape(B, 2)
